# baseline (speedup 1.0000x reference)
import jax
import jax.numpy as jnp
from jax.experimental import pallas as pl
from jax.experimental.pallas import tpu as pltpu

_GRID = 25


def _roll(x, shift):
    return jnp.roll(x, shift, axis=1)


def _body(t_ref, rot_ref, sc_ref, op_ref, xyz_ref, vel_ref, to_ref,
          fdc_ref, fr_ref,
          xyzt_ref, rotn_ref, scale_ref, opac_ref, feats_ref,
          sem1, sem2):
    i = pl.program_id(0)
    npts = fdc_ref.shape[0] // _GRID
    base = i * npts

    cp1 = pltpu.make_async_copy(
        fdc_ref.at[pl.ds(base, npts), :, :],
        feats_ref.at[pl.ds(base, npts), pl.ds(0, 1), :],
        sem1,
    )
    cp2 = pltpu.make_async_copy(
        fr_ref.at[pl.ds(base, npts), :, :],
        feats_ref.at[pl.ds(base, npts), pl.ds(1, 15), :],
        sem2,
    )
    cp1.start()
    cp2.start()

    scale_ref[...] = jnp.exp(sc_ref[...])
    opac_ref[...] = jax.nn.sigmoid(op_ref[...])

    r = rot_ref[...]
    x2 = r * r
    s = x2 + _roll(x2, -1) + _roll(x2, -2) + _roll(x2, -3)
    inv = jax.lax.rsqrt(jnp.maximum(s, 1e-24))
    lane = jax.lax.broadcasted_iota(jnp.int32, r.shape, 1)
    t0 = jnp.where((lane & 3) == 0, inv, 0.0)
    t1 = t0 + _roll(t0, 1)
    t2 = t1 + _roll(t1, 2)
    rotn_ref[...] = r * t2

    t = t_ref[0]
    dt = t - to_ref[...]
    ec = jax.lax.broadcasted_iota(jnp.int32, (200, 600), 1) // 3
    er = jax.lax.broadcasted_iota(jnp.int32, (200, 600), 0)
    expand = (ec == er).astype(jnp.float32)
    dte = jnp.dot(dt, expand, preferred_element_type=jnp.float32)
    xyzt_ref[...] = xyz_ref[...] + vel_ref[...] * dte

    cp1.wait()
    cp2.wait()


def kernel(xyz, rotation, scaling, opacity, features_dc, features_rest, time_offset, velocity, time):
    n = xyz.shape[0]
    t = jnp.asarray(time, jnp.float32).reshape(1)

    rot_in = rotation.reshape(n * 4 // 1000, 1000)
    sc_in = scaling.reshape(n * 3 // 1000, 1000)
    op_in = opacity.reshape(n // 1000, 1000)
    xyz_in = xyz.reshape(n * 3 // 600, 600)
    vel_in = velocity.reshape(n * 3 // 600, 600)
    to_in = time_offset.reshape(n // 200, 200)

    g = _GRID

    def rows(arr, d):
        return pl.BlockSpec((arr.shape[0] // g, d), lambda i: (i, 0))

    in_specs = [
        pl.BlockSpec(memory_space=pltpu.SMEM),
        rows(rot_in, 1000),
        rows(sc_in, 1000),
        rows(op_in, 1000),
        rows(xyz_in, 600),
        rows(vel_in, 600),
        rows(to_in, 200),
        pl.BlockSpec(memory_space=pl.ANY),
        pl.BlockSpec(memory_space=pl.ANY),
    ]
    out_specs = [
        rows(xyz_in, 600),
        rows(rot_in, 1000),
        rows(sc_in, 1000),
        rows(op_in, 1000),
        pl.BlockSpec(memory_space=pl.ANY),
    ]
    out_shape = [
        jax.ShapeDtypeStruct(xyz_in.shape, jnp.float32),
        jax.ShapeDtypeStruct(rot_in.shape, jnp.float32),
        jax.ShapeDtypeStruct(sc_in.shape, jnp.float32),
        jax.ShapeDtypeStruct(op_in.shape, jnp.float32),
        jax.ShapeDtypeStruct((n, 16, 3), jnp.float32),
    ]
    xyz_t, rot, scale, opac, feats = pl.pallas_call(
        _body,
        grid=(g,),
        in_specs=in_specs,
        out_specs=out_specs,
        out_shape=out_shape,
        scratch_shapes=[pltpu.SemaphoreType.DMA, pltpu.SemaphoreType.DMA],
        compiler_params=pltpu.CompilerParams(
            dimension_semantics=("arbitrary",),
        ),
    )(t, rot_in, sc_in, op_in, xyz_in, vel_in, to_in, features_dc, features_rest)
    return (
        xyz_t.reshape(n, 3),
        rot.reshape(n, 4),
        scale.reshape(n, 3),
        opac.reshape(n, 1),
        feats,
    )


# speedup vs baseline: 16.7852x; 16.7852x over previous
import jax
import jax.numpy as jnp
from jax.experimental import pallas as pl
from jax.experimental.pallas import tpu as pltpu

_GRID = 125


def _roll(x, shift):
    return jnp.roll(x, shift, axis=1)


def _body(t_ref, rot_ref, sc_ref, op_ref, xyz_ref, vel_ref, to_ref,
          fdc_ref, fr_ref,
          xyzt_ref, rotn_ref, scale_ref, opac_ref, feats_ref):
    lane48 = jax.lax.broadcasted_iota(jnp.int32, feats_ref.shape, 1)
    dc_w = jnp.pad(fdc_ref[...], ((0, 0), (0, 45)))
    rest_w = jnp.roll(jnp.pad(fr_ref[...], ((0, 0), (0, 3))), 3, axis=1)
    feats_ref[...] = jnp.where(lane48 < 3, dc_w, rest_w)

    scale_ref[...] = jnp.exp(sc_ref[...])
    opac_ref[...] = jax.nn.sigmoid(op_ref[...])

    r = rot_ref[...]
    x2 = r * r
    s = x2 + _roll(x2, -1) + _roll(x2, -2) + _roll(x2, -3)
    inv = jax.lax.rsqrt(jnp.maximum(s, 1e-24))
    lane = jax.lax.broadcasted_iota(jnp.int32, r.shape, 1)
    t0 = jnp.where((lane & 3) == 0, inv, 0.0)
    t1 = t0 + _roll(t0, 1)
    t2 = t1 + _roll(t1, 2)
    rotn_ref[...] = r * t2

    t = t_ref[0]
    dt = t - to_ref[...]
    ec = jax.lax.broadcasted_iota(jnp.int32, (200, 600), 1) // 3
    er = jax.lax.broadcasted_iota(jnp.int32, (200, 600), 0)
    expand = (ec == er).astype(jnp.float32)
    dte = jnp.dot(dt, expand, preferred_element_type=jnp.float32)
    xyzt_ref[...] = xyz_ref[...] + vel_ref[...] * dte


def kernel(xyz, rotation, scaling, opacity, features_dc, features_rest, time_offset, velocity, time):
    n = xyz.shape[0]
    t = jnp.asarray(time, jnp.float32).reshape(1)

    rot_in = rotation.reshape(n * 4 // 1000, 1000)
    sc_in = scaling.reshape(n * 3 // 1000, 1000)
    op_in = opacity.reshape(n // 1000, 1000)
    xyz_in = xyz.reshape(n * 3 // 600, 600)
    vel_in = velocity.reshape(n * 3 // 600, 600)
    to_in = time_offset.reshape(n // 200, 200)
    fdc_in = features_dc.reshape(n, 3)
    fr_in = features_rest.reshape(n, 45)
    feats_shape = jax.ShapeDtypeStruct((n, 48), jnp.float32)

    g = _GRID

    def rows(arr, d):
        return pl.BlockSpec((arr.shape[0] // g, d), lambda i: (i, 0))

    in_specs = [
        pl.BlockSpec(memory_space=pltpu.SMEM),
        rows(rot_in, 1000),
        rows(sc_in, 1000),
        rows(op_in, 1000),
        rows(xyz_in, 600),
        rows(vel_in, 600),
        rows(to_in, 200),
        rows(fdc_in, 3),
        rows(fr_in, 45),
    ]
    out_specs = [
        rows(xyz_in, 600),
        rows(rot_in, 1000),
        rows(sc_in, 1000),
        rows(op_in, 1000),
        rows(feats_shape, 48),
    ]
    out_shape = [
        jax.ShapeDtypeStruct(xyz_in.shape, jnp.float32),
        jax.ShapeDtypeStruct(rot_in.shape, jnp.float32),
        jax.ShapeDtypeStruct(sc_in.shape, jnp.float32),
        jax.ShapeDtypeStruct(op_in.shape, jnp.float32),
        feats_shape,
    ]
    xyz_t, rot, scale, opac, feats = pl.pallas_call(
        _body,
        grid=(g,),
        in_specs=in_specs,
        out_specs=out_specs,
        out_shape=out_shape,
        compiler_params=pltpu.CompilerParams(
            dimension_semantics=("arbitrary",),
        ),
    )(t, rot_in, sc_in, op_in, xyz_in, vel_in, to_in, fdc_in, fr_in)
    return (
        xyz_t.reshape(n, 3),
        rot.reshape(n, 4),
        scale.reshape(n, 3),
        opac.reshape(n, 1),
        feats.reshape(n, 16, 3),
    )


# speedup vs baseline: 1630.4093x; 97.1339x over previous
import jax
import jax.numpy as jnp
from jax.experimental import pallas as pl
from jax.experimental.pallas import tpu as pltpu

_LANES = 32768


def _body(t_ref, rot_ref, sc_ref, op_ref, xyz_ref, vel_ref, to_ref,
          fdc_ref, fr_ref,
          xyzt_ref, rotn_ref, scale_ref, opac_ref, feats_ref):
    r = rot_ref[...]
    s = jnp.sum(r * r, axis=0, keepdims=True)
    inv = jax.lax.rsqrt(jnp.maximum(s, 1e-24))
    rotn_ref[...] = r * inv

    scale_ref[...] = jnp.exp(sc_ref[...])
    opac_ref[...] = jax.nn.sigmoid(op_ref[...])

    dt = t_ref[0] - to_ref[...]
    xyzt_ref[...] = xyz_ref[...] + vel_ref[...] * dt

    feats_ref[:, 0, :] = fdc_ref[:, 0, :]
    feats_ref[:, 1:16, :] = fr_ref[...]


def kernel(xyz, rotation, scaling, opacity, features_dc, features_rest, time_offset, velocity, time):
    n = xyz.shape[0]
    t = jnp.asarray(time, jnp.float32).reshape(1)

    rot_t = rotation.T
    sc_t = scaling.T
    op_t = opacity.T
    xyz_t_in = xyz.T
    vel_t = velocity.T
    to_t = time_offset.T
    fdc_t = jnp.transpose(features_dc, (2, 1, 0))
    fr_t = jnp.transpose(features_rest, (2, 1, 0))

    L = _LANES
    g = pl.cdiv(n, L)

    def cols(d):
        return pl.BlockSpec((d, L), lambda i: (0, i))

    in_specs = [
        pl.BlockSpec(memory_space=pltpu.SMEM),
        cols(4),
        cols(3),
        cols(1),
        cols(3),
        cols(3),
        cols(1),
        pl.BlockSpec((3, 1, L), lambda i: (0, 0, i)),
        pl.BlockSpec((3, 15, L), lambda i: (0, 0, i)),
    ]
    out_specs = [
        cols(3),
        cols(4),
        cols(3),
        cols(1),
        pl.BlockSpec((3, 16, L), lambda i: (0, 0, i)),
    ]
    out_shape = [
        jax.ShapeDtypeStruct((3, n), jnp.float32),
        jax.ShapeDtypeStruct((4, n), jnp.float32),
        jax.ShapeDtypeStruct((3, n), jnp.float32),
        jax.ShapeDtypeStruct((1, n), jnp.float32),
        jax.ShapeDtypeStruct((3, 16, n), jnp.float32),
    ]
    xyzt_T, rotn_T, scale_T, opac_T, feats_T = pl.pallas_call(
        _body,
        grid=(g,),
        in_specs=in_specs,
        out_specs=out_specs,
        out_shape=out_shape,
        compiler_params=pltpu.CompilerParams(
            dimension_semantics=("arbitrary",),
        ),
    )(t, rot_t, sc_t, op_t, xyz_t_in, vel_t, to_t, fdc_t, fr_t)
    return (
        xyzt_T.T,
        rotn_T.T,
        scale_T.T,
        opac_T.T,
        jnp.transpose(feats_T, (2, 1, 0)),
    )


# speedup vs baseline: 1634.9177x; 1.0028x over previous
import jax
import jax.numpy as jnp
from jax.experimental import pallas as pl
from jax.experimental.pallas import tpu as pltpu

_LANES = 49152


def _body(t_ref, rot_ref, sc_ref, op_ref, xyz_ref, vel_ref, to_ref,
          fdc_ref, fr_ref,
          xyzt_ref, rotn_ref, scale_ref, opac_ref, feats_ref):
    r = rot_ref[...]
    s = jnp.sum(r * r, axis=0, keepdims=True)
    inv = jax.lax.rsqrt(jnp.maximum(s, 1e-24))
    rotn_ref[...] = r * inv

    scale_ref[...] = jnp.exp(sc_ref[...])
    opac_ref[...] = jax.nn.sigmoid(op_ref[...])

    dt = t_ref[0] - to_ref[...]
    xyzt_ref[...] = xyz_ref[...] + vel_ref[...] * dt

    feats_ref[:, 0, :] = fdc_ref[:, 0, :]
    feats_ref[:, 1:16, :] = fr_ref[...]


def kernel(xyz, rotation, scaling, opacity, features_dc, features_rest, time_offset, velocity, time):
    n = xyz.shape[0]
    t = jnp.asarray(time, jnp.float32).reshape(1)

    rot_t = rotation.T
    sc_t = scaling.T
    op_t = opacity.T
    xyz_t_in = xyz.T
    vel_t = velocity.T
    to_t = time_offset.T
    fdc_t = jnp.transpose(features_dc, (2, 1, 0))
    fr_t = jnp.transpose(features_rest, (2, 1, 0))

    L = _LANES
    g = pl.cdiv(n, L)

    def cols(d):
        return pl.BlockSpec((d, L), lambda i: (0, i))

    in_specs = [
        pl.BlockSpec(memory_space=pltpu.SMEM),
        cols(4),
        cols(3),
        cols(1),
        cols(3),
        cols(3),
        cols(1),
        pl.BlockSpec((3, 1, L), lambda i: (0, 0, i)),
        pl.BlockSpec((3, 15, L), lambda i: (0, 0, i)),
    ]
    out_specs = [
        cols(3),
        cols(4),
        cols(3),
        cols(1),
        pl.BlockSpec((3, 16, L), lambda i: (0, 0, i)),
    ]
    out_shape = [
        jax.ShapeDtypeStruct((3, n), jnp.float32),
        jax.ShapeDtypeStruct((4, n), jnp.float32),
        jax.ShapeDtypeStruct((3, n), jnp.float32),
        jax.ShapeDtypeStruct((1, n), jnp.float32),
        jax.ShapeDtypeStruct((3, 16, n), jnp.float32),
    ]
    xyzt_T, rotn_T, scale_T, opac_T, feats_T = pl.pallas_call(
        _body,
        grid=(g,),
        in_specs=in_specs,
        out_specs=out_specs,
        out_shape=out_shape,
        compiler_params=pltpu.CompilerParams(
            dimension_semantics=("arbitrary",),
        ),
    )(t, rot_t, sc_t, op_t, xyz_t_in, vel_t, to_t, fdc_t, fr_t)
    return (
        xyzt_T.T,
        rotn_T.T,
        scale_T.T,
        opac_T.T,
        jnp.transpose(feats_T, (2, 1, 0)),
    )
